# baseline (speedup 1.0000x reference)
import functools

import jax
import jax.numpy as jnp
from jax import lax
from jax.experimental import pallas as pl
from jax.experimental.pallas import tpu as pltpu
from jax.experimental.pallas import tpu_sc as plsc

D = 128
P = 4
N = 16
PN = P * N

_NW = 32
_CHUNK = 128
_T = 128
_S = 8


def _sc_gather(item_table, user_table, nbi_idx, nbu_idx, ci_idx, cu_idx):
    nn = nbi_idx.shape[0]
    nc = ci_idx.shape[0]
    per_w = nn // _NW
    n_chunks = per_w // _CHUNK
    cper_w = nc // _NW
    mesh = plsc.VectorSubcoreMesh(core_axis_name="c", subcore_axis_name="s")

    @functools.partial(
        pl.kernel,
        mesh=mesh,
        out_type=[
            jax.ShapeDtypeStruct((nn, D), jnp.float32),
            jax.ShapeDtypeStruct((nn, D), jnp.float32),
            jax.ShapeDtypeStruct((nc, D), jnp.float32),
            jax.ShapeDtypeStruct((nc, D), jnp.float32),
        ],
        scratch_types=[
            pltpu.VMEM((per_w,), jnp.int32),
            pltpu.VMEM((_CHUNK, D), jnp.float32),
            pltpu.VMEM((_CHUNK, D), jnp.float32),
            pltpu.VMEM((_CHUNK, D), jnp.float32),
            pltpu.VMEM((_CHUNK, D), jnp.float32),
            pltpu.VMEM((cper_w,), jnp.int32),
            pltpu.VMEM((cper_w, D), jnp.float32),
            pltpu.SemaphoreType.DMA,
            pltpu.SemaphoreType.DMA,
            pltpu.SemaphoreType.DMA,
        ],
    )
    def gather_k(item_hbm, user_hbm,
                 nbi_hbm, nbu_hbm, ci_hbm, cu_hbm,
                 out_nbi, out_nbu, out_ci, out_cu,
                 idx_all, rows0, rows1, rows2, rows3, cidx, crows,
                 gsem, wsem, csem):
        wid = lax.axis_index("s") * 2 + lax.axis_index("c")
        base = wid * per_w
        cbase = wid * cper_w
        rows = (rows0, rows1, rows2, rows3)
        depth = 3

        def run(table_hbm, nbidx_hbm, cidx_hbm, out_nb, out_c):
            pltpu.sync_copy(nbidx_hbm.at[pl.ds(pl.multiple_of(base, _CHUNK),
                                               per_w)], idx_all)
            pltpu.sync_copy(cidx_hbm.at[pl.ds(pl.multiple_of(cbase, 8),
                                              cper_w)], cidx)

            def gath(k):
                off = pl.multiple_of(k * _CHUNK, _CHUNK)
                pltpu.async_copy(table_hbm.at[idx_all.at[pl.ds(off, _CHUNK)]],
                                 rows[k % 4], gsem)

            def out_at(k):
                off = pl.multiple_of(base + k * _CHUNK, _CHUNK)
                return out_nb.at[pl.ds(off, _CHUNK)]

            for k in range(depth):
                gath(k)
            pltpu.async_copy(table_hbm.at[cidx], crows, csem)

            n_wsem_waits = 0
            for k in range(n_chunks):
                pltpu.make_async_copy(
                    table_hbm.at[idx_all.at[pl.ds(0, _CHUNK)]],
                    rows[k % 4], gsem).wait()
                pltpu.async_copy(rows[k % 4], out_at(k), wsem)
                nxt = k + depth
                if nxt < n_chunks:
                    if k >= 1:
                        pltpu.make_async_copy(rows[0], out_at(0), wsem).wait()
                        n_wsem_waits += 1
                    gath(nxt)
            for _ in range(n_chunks - n_wsem_waits):
                pltpu.make_async_copy(rows[0], out_at(0), wsem).wait()

            pltpu.make_async_copy(table_hbm.at[cidx], crows, csem).wait()
            pltpu.sync_copy(crows,
                            out_c.at[pl.ds(pl.multiple_of(cbase, 8), cper_w)])

        run(item_hbm, nbi_hbm, ci_hbm, out_nbi, out_ci)
        run(user_hbm, nbu_hbm, cu_hbm, out_nbu, out_cu)

    return gather_k(item_table, user_table, nbi_idx, nbu_idx, ci_idx, cu_idx)


def _attn_body(nbu_ref, nbv_ref, xu_ref, xv_ref, Wu_ref, au_ref, Wi_ref,
               ai_ref, qu_ref, qi_ref, Wf_ref, out_ref):
    def side(nb3, x, W, Arep, Qrep):
        h = jnp.dot(nb3.reshape(PN * _T, D), W,
                    preferred_element_type=jnp.float32)
        xq = jnp.dot(x, W, preferred_element_type=jnp.float32)
        t = jnp.tanh(h.reshape(PN, _T, D) + xq[None]).reshape(PN * _T, D)
        e = jnp.exp(jnp.dot(t, Arep, preferred_element_type=jnp.float32))
        num = (e * h).reshape(P, N, _T, D).sum(axis=1)
        den = e.reshape(P, N, _T, D).sum(axis=1)
        pe = num / den
        ep = jnp.exp(jnp.dot(jnp.tanh(pe).reshape(P * _T, D), Qrep,
                             preferred_element_type=jnp.float32))
        ep = ep.reshape(P, _T, D)
        aggn = (ep * pe).sum(axis=0)
        aggd = ep.sum(axis=0)
        return x + aggn / aggd

    u = side(nbu_ref[...], xu_ref[...], Wu_ref[...], au_ref[...], qu_ref[...])
    v = side(nbv_ref[...], xv_ref[...], Wi_ref[...], ai_ref[...], qi_ref[...])
    vf = jnp.dot(v, Wf_ref[...], preferred_element_type=jnp.float32)
    out_ref[...] = jnp.sum(u * vf, axis=-1, keepdims=True)


def _tc_attn(nb_u, nb_v, x_u, x_v, Wu, Au, Wi, Ai, Qu, Qi, Wf):
    Bs = x_u.shape[0]
    grid = Bs // _T
    nb_u = nb_u.reshape(PN, Bs, D)
    nb_v = nb_v.reshape(PN, Bs, D)
    nb_spec = pl.BlockSpec((PN, _T, D), lambda i: (0, i, 0))
    ctr_spec = pl.BlockSpec((_T, D), lambda i: (i, 0))
    w_spec = pl.BlockSpec((D, D), lambda i: (0, 0))
    return pl.pallas_call(
        _attn_body,
        grid=(grid,),
        in_specs=[nb_spec, nb_spec, ctr_spec, ctr_spec,
                  w_spec, w_spec, w_spec, w_spec,
                  w_spec, w_spec, w_spec],
        out_specs=pl.BlockSpec((_T, 1), lambda i: (i, 0)),
        out_shape=jax.ShapeDtypeStruct((Bs, 1), jnp.float32),
    )(nb_u, nb_v, x_u, x_v, Wu, Au, Wi, Ai, Qu, Qi, Wf)


def kernel(user_table, item_table, W_homo_u, a_homo_u, W_homo_i, a_homo_i,
           q_hete_u, q_hete_i, W_fuse, user_ids, item_ids, user_neighs,
           item_neighs):
    B = user_ids.shape[0]
    Bs = B // _S
    un = user_neighs.astype(jnp.int32).transpose(1, 2, 0)
    itn = item_neighs.astype(jnp.int32).transpose(1, 2, 0)
    uid = user_ids.astype(jnp.int32)
    iid = item_ids.astype(jnp.int32)
    ones = jnp.ones((1, D), jnp.float32)
    Au = a_homo_u[:, None] * ones
    Ai = a_homo_i[:, None] * ones
    Qu = q_hete_u[:, None] * ones
    Qi = q_hete_i[:, None] * ones

    outs = []
    for s in range(_S):
        sl = slice(s * Bs, (s + 1) * Bs)
        g_nbi, g_nbu, g_ci, g_cu = _sc_gather(
            item_table, user_table,
            un[:, :, sl].reshape(-1), itn[:, :, sl].reshape(-1),
            iid[sl], uid[sl])
        outs.append(_tc_attn(g_nbi, g_nbu, g_cu, g_ci,
                             W_homo_u, Au, W_homo_i, Ai, Qu, Qi, W_fuse))
    return jnp.concatenate(outs, axis=0).reshape(B)


# speedup vs baseline: 1.0422x; 1.0422x over previous
import functools

import jax
import jax.numpy as jnp
from jax import lax
from jax.experimental import pallas as pl
from jax.experimental.pallas import tpu as pltpu
from jax.experimental.pallas import tpu_sc as plsc

D = 128
P = 4
N = 16
PN = P * N

_NW = 32
_CHUNK = 128
_T = 128
_S = 4


def _sc_gather(item_table, user_table, nbi_idx, nbu_idx, ci_idx, cu_idx):
    nn = nbi_idx.shape[0]
    nc = ci_idx.shape[0]
    per_w = nn // _NW
    n_chunks = per_w // _CHUNK
    cper_w = nc // _NW
    mesh = plsc.VectorSubcoreMesh(core_axis_name="c", subcore_axis_name="s")

    @functools.partial(
        pl.kernel,
        mesh=mesh,
        out_type=[
            jax.ShapeDtypeStruct((nn, D), jnp.float32),
            jax.ShapeDtypeStruct((nn, D), jnp.float32),
            jax.ShapeDtypeStruct((nc, D), jnp.float32),
            jax.ShapeDtypeStruct((nc, D), jnp.float32),
        ],
        scratch_types=[
            pltpu.VMEM((per_w,), jnp.int32),
            pltpu.VMEM((_CHUNK, D), jnp.float32),
            pltpu.VMEM((_CHUNK, D), jnp.float32),
            pltpu.VMEM((_CHUNK, D), jnp.float32),
            pltpu.VMEM((_CHUNK, D), jnp.float32),
            pltpu.VMEM((_CHUNK, D), jnp.float32),
            pltpu.VMEM((_CHUNK, D), jnp.float32),
            pltpu.VMEM((cper_w,), jnp.int32),
            pltpu.VMEM((cper_w, D), jnp.float32),
            pltpu.SemaphoreType.DMA,
            pltpu.SemaphoreType.DMA,
            pltpu.SemaphoreType.DMA,
        ],
    )
    def gather_k(item_hbm, user_hbm,
                 nbi_hbm, nbu_hbm, ci_hbm, cu_hbm,
                 out_nbi, out_nbu, out_ci, out_cu,
                 idx_all, rows0, rows1, rows2, rows3, rows4, rows5,
                 cidx, crows, gsem, wsem, csem):
        wid = lax.axis_index("s") * 2 + lax.axis_index("c")
        base = wid * per_w
        cbase = wid * cper_w
        rows = (rows0, rows1, rows2, rows3, rows4, rows5)
        nbuf = 6
        depth = 5

        def run(table_hbm, nbidx_hbm, cidx_hbm, out_nb, out_c):
            pltpu.sync_copy(nbidx_hbm.at[pl.ds(pl.multiple_of(base, _CHUNK),
                                               per_w)], idx_all)
            pltpu.sync_copy(cidx_hbm.at[pl.ds(pl.multiple_of(cbase, 8),
                                              cper_w)], cidx)

            def gath(k):
                off = pl.multiple_of(k * _CHUNK, _CHUNK)
                pltpu.async_copy(table_hbm.at[idx_all.at[pl.ds(off, _CHUNK)]],
                                 rows[k % nbuf], gsem)

            def out_at(k):
                off = pl.multiple_of(base + k * _CHUNK, _CHUNK)
                return out_nb.at[pl.ds(off, _CHUNK)]

            for k in range(depth):
                gath(k)
            pltpu.async_copy(table_hbm.at[cidx], crows, csem)

            n_wsem_waits = 0
            for k in range(n_chunks):
                pltpu.make_async_copy(
                    table_hbm.at[idx_all.at[pl.ds(0, _CHUNK)]],
                    rows[k % nbuf], gsem).wait()
                pltpu.async_copy(rows[k % nbuf], out_at(k), wsem)
                nxt = k + depth
                if nxt < n_chunks:
                    if k >= 1:
                        pltpu.make_async_copy(rows[0], out_at(0), wsem).wait()
                        n_wsem_waits += 1
                    gath(nxt)
            for _ in range(n_chunks - n_wsem_waits):
                pltpu.make_async_copy(rows[0], out_at(0), wsem).wait()

            pltpu.make_async_copy(table_hbm.at[cidx], crows, csem).wait()
            pltpu.sync_copy(crows,
                            out_c.at[pl.ds(pl.multiple_of(cbase, 8), cper_w)])

        run(item_hbm, nbi_hbm, ci_hbm, out_nbi, out_ci)
        run(user_hbm, nbu_hbm, cu_hbm, out_nbu, out_cu)

    return gather_k(item_table, user_table, nbi_idx, nbu_idx, ci_idx, cu_idx)


def _attn_body(nbu_ref, nbv_ref, xu_ref, xv_ref, Wu_ref, au_ref, Wi_ref,
               ai_ref, qu_ref, qi_ref, Wf_ref, out_ref):
    def side(nb3, x, W, Arep, Qrep):
        h = jnp.dot(nb3.reshape(PN * _T, D), W,
                    preferred_element_type=jnp.float32)
        xq = jnp.dot(x, W, preferred_element_type=jnp.float32)
        t = jnp.tanh(h.reshape(PN, _T, D) + xq[None]).reshape(PN * _T, D)
        e = jnp.exp(jnp.dot(t, Arep, preferred_element_type=jnp.float32))
        num = (e * h).reshape(P, N, _T, D).sum(axis=1)
        den = e.reshape(P, N, _T, D).sum(axis=1)
        pe = num / den
        ep = jnp.exp(jnp.dot(jnp.tanh(pe).reshape(P * _T, D), Qrep,
                             preferred_element_type=jnp.float32))
        ep = ep.reshape(P, _T, D)
        aggn = (ep * pe).sum(axis=0)
        aggd = ep.sum(axis=0)
        return x + aggn / aggd

    u = side(nbu_ref[...], xu_ref[...], Wu_ref[...], au_ref[...], qu_ref[...])
    v = side(nbv_ref[...], xv_ref[...], Wi_ref[...], ai_ref[...], qi_ref[...])
    vf = jnp.dot(v, Wf_ref[...], preferred_element_type=jnp.float32)
    out_ref[...] = jnp.sum(u * vf, axis=-1, keepdims=True)


def _tc_attn(nb_u, nb_v, x_u, x_v, Wu, Au, Wi, Ai, Qu, Qi, Wf):
    Bs = x_u.shape[0]
    grid = Bs // _T
    nb_u = nb_u.reshape(PN, Bs, D)
    nb_v = nb_v.reshape(PN, Bs, D)
    nb_spec = pl.BlockSpec((PN, _T, D), lambda i: (0, i, 0))
    ctr_spec = pl.BlockSpec((_T, D), lambda i: (i, 0))
    w_spec = pl.BlockSpec((D, D), lambda i: (0, 0))
    return pl.pallas_call(
        _attn_body,
        grid=(grid,),
        in_specs=[nb_spec, nb_spec, ctr_spec, ctr_spec,
                  w_spec, w_spec, w_spec, w_spec,
                  w_spec, w_spec, w_spec],
        out_specs=pl.BlockSpec((_T, 1), lambda i: (i, 0)),
        out_shape=jax.ShapeDtypeStruct((Bs, 1), jnp.float32),
    )(nb_u, nb_v, x_u, x_v, Wu, Au, Wi, Ai, Qu, Qi, Wf)


def kernel(user_table, item_table, W_homo_u, a_homo_u, W_homo_i, a_homo_i,
           q_hete_u, q_hete_i, W_fuse, user_ids, item_ids, user_neighs,
           item_neighs):
    B = user_ids.shape[0]
    Bs = B // _S
    un = user_neighs.astype(jnp.int32).transpose(1, 2, 0)
    itn = item_neighs.astype(jnp.int32).transpose(1, 2, 0)
    uid = user_ids.astype(jnp.int32)
    iid = item_ids.astype(jnp.int32)
    ones = jnp.ones((1, D), jnp.float32)
    Au = a_homo_u[:, None] * ones
    Ai = a_homo_i[:, None] * ones
    Qu = q_hete_u[:, None] * ones
    Qi = q_hete_i[:, None] * ones

    outs = []
    for s in range(_S):
        sl = slice(s * Bs, (s + 1) * Bs)
        g_nbi, g_nbu, g_ci, g_cu = _sc_gather(
            item_table, user_table,
            un[:, :, sl].reshape(-1), itn[:, :, sl].reshape(-1),
            iid[sl], uid[sl])
        outs.append(_tc_attn(g_nbi, g_nbu, g_cu, g_ci,
                             W_homo_u, Au, W_homo_i, Ai, Qu, Qi, W_fuse))
    return jnp.concatenate(outs, axis=0).reshape(B)


# speedup vs baseline: 1.0490x; 1.0065x over previous
import functools

import jax
import jax.numpy as jnp
from jax import lax
from jax.experimental import pallas as pl
from jax.experimental.pallas import tpu as pltpu
from jax.experimental.pallas import tpu_sc as plsc

D = 128
P = 4
N = 16
PN = P * N

_NW = 32
_CHUNK = 128
_T = 128
_SLICES = (512, 1536, 1536, 512)


def _sc_gather(item_table, user_table, nbi_idx, nbu_idx, ci_idx, cu_idx):
    nn = nbi_idx.shape[0]
    nc = ci_idx.shape[0]
    per_w = nn // _NW
    n_chunks = per_w // _CHUNK
    cper_w = nc // _NW
    mesh = plsc.VectorSubcoreMesh(core_axis_name="c", subcore_axis_name="s")

    @functools.partial(
        pl.kernel,
        mesh=mesh,
        out_type=[
            jax.ShapeDtypeStruct((nn, D), jnp.float32),
            jax.ShapeDtypeStruct((nn, D), jnp.float32),
            jax.ShapeDtypeStruct((nc, D), jnp.float32),
            jax.ShapeDtypeStruct((nc, D), jnp.float32),
        ],
        scratch_types=[
            pltpu.VMEM((per_w,), jnp.int32),
            pltpu.VMEM((_CHUNK, D), jnp.float32),
            pltpu.VMEM((_CHUNK, D), jnp.float32),
            pltpu.VMEM((_CHUNK, D), jnp.float32),
            pltpu.VMEM((_CHUNK, D), jnp.float32),
            pltpu.VMEM((cper_w,), jnp.int32),
            pltpu.VMEM((cper_w, D), jnp.float32),
            pltpu.SemaphoreType.DMA,
            pltpu.SemaphoreType.DMA,
            pltpu.SemaphoreType.DMA,
        ],
    )
    def gather_k(item_hbm, user_hbm,
                 nbi_hbm, nbu_hbm, ci_hbm, cu_hbm,
                 out_nbi, out_nbu, out_ci, out_cu,
                 idx_all, rows0, rows1, rows2, rows3,
                 cidx, crows, gsem, wsem, csem):
        wid = lax.axis_index("s") * 2 + lax.axis_index("c")
        base = wid * per_w
        cbase = wid * cper_w
        rows = (rows0, rows1, rows2, rows3)
        nbuf = 4
        depth = 3

        def run(table_hbm, nbidx_hbm, cidx_hbm, out_nb, out_c):
            pltpu.sync_copy(nbidx_hbm.at[pl.ds(pl.multiple_of(base, _CHUNK),
                                               per_w)], idx_all)
            pltpu.sync_copy(cidx_hbm.at[pl.ds(pl.multiple_of(cbase, 8),
                                              cper_w)], cidx)

            def gath(k):
                off = pl.multiple_of(k * _CHUNK, _CHUNK)
                pltpu.async_copy(table_hbm.at[idx_all.at[pl.ds(off, _CHUNK)]],
                                 rows[k % nbuf], gsem)

            def out_at(k):
                off = pl.multiple_of(base + k * _CHUNK, _CHUNK)
                return out_nb.at[pl.ds(off, _CHUNK)]

            for k in range(depth):
                gath(k)
            pltpu.async_copy(table_hbm.at[cidx], crows, csem)

            n_wsem_waits = 0
            for k in range(n_chunks):
                pltpu.make_async_copy(
                    table_hbm.at[idx_all.at[pl.ds(0, _CHUNK)]],
                    rows[k % nbuf], gsem).wait()
                pltpu.async_copy(rows[k % nbuf], out_at(k), wsem)
                nxt = k + depth
                if nxt < n_chunks:
                    if k >= 1:
                        pltpu.make_async_copy(rows[0], out_at(0), wsem).wait()
                        n_wsem_waits += 1
                    gath(nxt)
            for _ in range(n_chunks - n_wsem_waits):
                pltpu.make_async_copy(rows[0], out_at(0), wsem).wait()

            pltpu.make_async_copy(table_hbm.at[cidx], crows, csem).wait()
            pltpu.sync_copy(crows,
                            out_c.at[pl.ds(pl.multiple_of(cbase, 8), cper_w)])

        run(item_hbm, nbi_hbm, ci_hbm, out_nbi, out_ci)
        run(user_hbm, nbu_hbm, cu_hbm, out_nbu, out_cu)

    return gather_k(item_table, user_table, nbi_idx, nbu_idx, ci_idx, cu_idx)


def _attn_body(nbu_ref, nbv_ref, xu_ref, xv_ref, Wu_ref, au_ref, Wi_ref,
               ai_ref, qu_ref, qi_ref, Wf_ref, out_ref):
    def side(nb3, x, W, Arep, Qrep):
        h = jnp.dot(nb3.reshape(PN * _T, D), W,
                    preferred_element_type=jnp.float32)
        xq = jnp.dot(x, W, preferred_element_type=jnp.float32)
        t = jnp.tanh(h.reshape(PN, _T, D) + xq[None]).reshape(PN * _T, D)
        e = jnp.exp(jnp.dot(t, Arep, preferred_element_type=jnp.float32))
        num = (e * h).reshape(P, N, _T, D).sum(axis=1)
        den = e.reshape(P, N, _T, D).sum(axis=1)
        pe = num / den
        ep = jnp.exp(jnp.dot(jnp.tanh(pe).reshape(P * _T, D), Qrep,
                             preferred_element_type=jnp.float32))
        ep = ep.reshape(P, _T, D)
        aggn = (ep * pe).sum(axis=0)
        aggd = ep.sum(axis=0)
        return x + aggn / aggd

    u = side(nbu_ref[...], xu_ref[...], Wu_ref[...], au_ref[...], qu_ref[...])
    v = side(nbv_ref[...], xv_ref[...], Wi_ref[...], ai_ref[...], qi_ref[...])
    vf = jnp.dot(v, Wf_ref[...], preferred_element_type=jnp.float32)
    out_ref[...] = jnp.sum(u * vf, axis=-1, keepdims=True)


def _tc_attn(nb_u, nb_v, x_u, x_v, Wu, Au, Wi, Ai, Qu, Qi, Wf):
    Bs = x_u.shape[0]
    grid = Bs // _T
    nb_u = nb_u.reshape(PN, Bs, D)
    nb_v = nb_v.reshape(PN, Bs, D)
    nb_spec = pl.BlockSpec((PN, _T, D), lambda i: (0, i, 0))
    ctr_spec = pl.BlockSpec((_T, D), lambda i: (i, 0))
    w_spec = pl.BlockSpec((D, D), lambda i: (0, 0))
    return pl.pallas_call(
        _attn_body,
        grid=(grid,),
        in_specs=[nb_spec, nb_spec, ctr_spec, ctr_spec,
                  w_spec, w_spec, w_spec, w_spec,
                  w_spec, w_spec, w_spec],
        out_specs=pl.BlockSpec((_T, 1), lambda i: (i, 0)),
        out_shape=jax.ShapeDtypeStruct((Bs, 1), jnp.float32),
    )(nb_u, nb_v, x_u, x_v, Wu, Au, Wi, Ai, Qu, Qi, Wf)


def kernel(user_table, item_table, W_homo_u, a_homo_u, W_homo_i, a_homo_i,
           q_hete_u, q_hete_i, W_fuse, user_ids, item_ids, user_neighs,
           item_neighs):
    B = user_ids.shape[0]
    un = user_neighs.astype(jnp.int32).transpose(1, 2, 0)
    itn = item_neighs.astype(jnp.int32).transpose(1, 2, 0)
    uid = user_ids.astype(jnp.int32)
    iid = item_ids.astype(jnp.int32)
    ones = jnp.ones((1, D), jnp.float32)
    Au = a_homo_u[:, None] * ones
    Ai = a_homo_i[:, None] * ones
    Qu = q_hete_u[:, None] * ones
    Qi = q_hete_i[:, None] * ones

    outs = []
    off = 0
    for bs in _SLICES:
        sl = slice(off, off + bs)
        off += bs
        g_nbi, g_nbu, g_ci, g_cu = _sc_gather(
            item_table, user_table,
            un[:, :, sl].reshape(-1), itn[:, :, sl].reshape(-1),
            iid[sl], uid[sl])
        outs.append(_tc_attn(g_nbi, g_nbu, g_cu, g_ci,
                             W_homo_u, Au, W_homo_i, Ai, Qu, Qi, W_fuse))
    return jnp.concatenate(outs, axis=0).reshape(B)
